# baseline (speedup 1.0000x reference)
import functools

import jax
import jax.numpy as jnp
from jax import lax
from jax.experimental import pallas as pl
from jax.experimental.pallas import tpu as pltpu
from jax.experimental.pallas import tpu_sc as plsc

NC = 2
NS = 16
NW = NC * NS
CHUNK = 96

B_SEG = 64


def _make_sc_agg(n_rows, d, nchunk):
    rpt = ((-(-(n_rows + 1) // NS)) + 7) // 8 * 8
    npad = rpt * NS
    full, rem = divmod(rpt, CHUNK)

    mesh = plsc.VectorSubcoreMesh(
        core_axis_name="c", subcore_axis_name="s",
        num_cores=NC, num_subcores=NS)

    @functools.partial(
        pl.kernel,
        out_type=jax.ShapeDtypeStruct((NC, npad, d), jnp.float32),
        mesh=mesh,
        scratch_types=[
            pltpu.VMEM((nchunk, CHUNK), jnp.int32),
            pltpu.VMEM((nchunk, CHUNK), jnp.int32),
            pltpu.VMEM((CHUNK, d), jnp.float32),
            pltpu.VMEM((CHUNK, d), jnp.float32),
            pltpu.VMEM_SHARED((npad, d), jnp.float32),
            pltpu.SemaphoreType.DMA,
            pltpu.SemaphoreType.DMA,
        ],
        compiler_params=pltpu.CompilerParams(use_tc_tiling_on_sc=False),
    )
    def sc_agg(h_hbm, src_hbm, dst_hbm, out_hbm, idx_s, idx_d,
               rows0, rows1, acc, sem0, sem1):
        cid = lax.axis_index("c")
        sid = lax.axis_index("s")
        wid = sid * NC + cid

        def zero_body(i, _):
            r = i // (d // 16)
            col = (i % (d // 16)) * 16
            rows0[r, pl.ds(col, 16)] = jnp.zeros((16,), jnp.float32)
            return 0
        lax.fori_loop(0, CHUNK * d // 16, zero_body, 0)
        zbase = sid * rpt
        for k in range(full):
            pltpu.sync_copy(rows0, acc.at[pl.ds(zbase + k * CHUNK, CHUNK)])
        if rem:
            pltpu.sync_copy(rows0.at[pl.ds(0, rem)],
                            acc.at[pl.ds(zbase + full * CHUNK, rem)])
        plsc.subcore_barrier()

        pltpu.sync_copy(src_hbm.at[wid], idx_s)
        pltpu.sync_copy(dst_hbm.at[wid], idx_d)

        npairs = nchunk // 2
        pltpu.async_copy(h_hbm.at[idx_s.at[0]], rows0, sem0)

        def pair_body(p, _):
            c0 = 2 * p
            pltpu.async_copy(h_hbm.at[idx_s.at[c0 + 1]], rows1, sem1)
            pltpu.make_async_copy(h_hbm.at[idx_s.at[c0]], rows0, sem0).wait()
            pltpu.sync_copy(rows0, acc.at[idx_d.at[c0]], add=True)

            @pl.when(p + 1 < npairs)
            def _():
                pltpu.async_copy(h_hbm.at[idx_s.at[c0 + 2]], rows0, sem0)
            pltpu.make_async_copy(h_hbm.at[idx_s.at[c0 + 1]], rows1, sem1).wait()
            pltpu.sync_copy(rows1, acc.at[idx_d.at[c0 + 1]], add=True)
            return 0
        lax.fori_loop(0, npairs, pair_body, 0)
        plsc.subcore_barrier()

        pltpu.sync_copy(acc.at[pl.ds(zbase, rpt)],
                        out_hbm.at[cid, pl.ds(zbase, rpt)])

    return sc_agg, npad


def _pick_bm(n):
    for bm in (512, 400, 256, 200, 128, 80, 40, 16, 8):
        if n % bm == 0:
            return bm
    return n


def _mlp_body(h_ref, a0_ref, a1_ref, w1_ref, b1_ref, w2_ref, b2_ref, o_ref):
    hs = h_ref[...] + a0_ref[...] + a1_ref[...]
    z = jnp.dot(hs, w1_ref[...], preferred_element_type=jnp.float32) + b1_ref[...]
    z = jnp.maximum(z, 0.0)
    z = jnp.dot(z, w2_ref[...], preferred_element_type=jnp.float32) + b2_ref[...]
    o_ref[...] = jnp.maximum(z, 0.0)


def _mlp_layer(h, a0, a1, w1, b1, w2, b2):
    n, din = h.shape
    hdim = w1.shape[1]
    bm = _pick_bm(n)
    return pl.pallas_call(
        _mlp_body,
        grid=(n // bm,),
        in_specs=[
            pl.BlockSpec((bm, din), lambda i: (i, 0)),
            pl.BlockSpec((bm, din), lambda i: (i, 0)),
            pl.BlockSpec((bm, din), lambda i: (i, 0)),
            pl.BlockSpec((din, hdim), lambda i: (0, 0)),
            pl.BlockSpec((1, hdim), lambda i: (0, 0)),
            pl.BlockSpec((hdim, hdim), lambda i: (0, 0)),
            pl.BlockSpec((1, hdim), lambda i: (0, 0)),
        ],
        out_specs=pl.BlockSpec((bm, hdim), lambda i: (i, 0)),
        out_shape=jax.ShapeDtypeStruct((n, hdim), jnp.float32),
    )(h, a0, a1, w1, b1.reshape(1, -1), w2, b2.reshape(1, -1))


def _final_layer(h, a0, a1, w1, b1, w2, b2, lin_w, lin_b, batch_i32):
    n, din = h.shape
    hdim = w1.shape[1]
    bm = _pick_bm(n)

    def body(h_ref, a0_ref, a1_ref, w1_ref, b1_ref, w2_ref, b2_ref,
             lw_ref, lb_ref, bt_ref, o_ref):
        hs = h_ref[...] + a0_ref[...] + a1_ref[...]
        z = jnp.dot(hs, w1_ref[...], preferred_element_type=jnp.float32) + b1_ref[...]
        z = jnp.maximum(z, 0.0)
        z = jnp.dot(z, w2_ref[...], preferred_element_type=jnp.float32) + b2_ref[...]
        hh = jnp.maximum(z, 0.0)
        y = jnp.dot(hh, lw_ref[...], preferred_element_type=jnp.float32)
        seg = lax.broadcasted_iota(jnp.int32, (bm, B_SEG), 1)
        oh = (bt_ref[...] == seg).astype(jnp.float32)
        contrib = jnp.sum(oh * y, axis=0, keepdims=True)

        @pl.when(pl.program_id(0) == 0)
        def _():
            o_ref[...] = contrib + lb_ref[...]

        @pl.when(pl.program_id(0) != 0)
        def _():
            o_ref[...] = o_ref[...] + contrib

    out = pl.pallas_call(
        body,
        grid=(n // bm,),
        in_specs=[
            pl.BlockSpec((bm, din), lambda i: (i, 0)),
            pl.BlockSpec((bm, din), lambda i: (i, 0)),
            pl.BlockSpec((bm, din), lambda i: (i, 0)),
            pl.BlockSpec((din, hdim), lambda i: (0, 0)),
            pl.BlockSpec((1, hdim), lambda i: (0, 0)),
            pl.BlockSpec((hdim, hdim), lambda i: (0, 0)),
            pl.BlockSpec((1, hdim), lambda i: (0, 0)),
            pl.BlockSpec((hdim, 1), lambda i: (0, 0)),
            pl.BlockSpec((1, 1), lambda i: (0, 0)),
            pl.BlockSpec((bm, 1), lambda i: (i, 0)),
        ],
        out_specs=pl.BlockSpec((1, B_SEG), lambda i: (0, 0)),
        out_shape=jax.ShapeDtypeStruct((1, B_SEG), jnp.float32),
    )(h, a0, a1, w1, b1.reshape(1, -1), w2, b2.reshape(1, -1),
      lin_w, lin_b.reshape(1, 1), batch_i32)
    return out[0]


def kernel(x, edge_index, edge_attr, batch,
           W1_0, b1_0, W2_0, b2_0, W1_1, b1_1, W2_1, b2_1,
           W1_2, b1_2, W2_2, b2_2, lin_W, lin_b):
    n, d = x.shape
    e = edge_index.shape[1]
    nchunk = -(-e // (NW * CHUNK))
    nchunk += nchunk % 2
    e_pad = NW * nchunk * CHUNK

    src = edge_index[0]
    dst = edge_index[1]
    if e_pad > e:
        src = jnp.concatenate([src, jnp.zeros((e_pad - e,), jnp.int32)])
        dst = jnp.concatenate([dst, jnp.full((e_pad - e,), n, jnp.int32)])
    src = src.reshape(NW, nchunk, CHUNK)
    dst = dst.reshape(NW, nchunk, CHUNK)

    batch_i32 = batch.astype(jnp.int32).reshape(n, 1)

    layers = [(W1_0, b1_0, W2_0, b2_0), (W1_1, b1_1, W2_1, b2_1),
              (W1_2, b1_2, W2_2, b2_2)]

    h = x
    for li, (w1, bb1, w2, bb2) in enumerate(layers):
        sc_agg, npad = _make_sc_agg(n, h.shape[1], nchunk)
        agg = sc_agg(h, src, dst)
        a0 = agg[0, :n]
        a1 = agg[1, :n]
        if li < 2:
            h = _mlp_layer(h, a0, a1, w1, bb1, w2, bb2)
        else:
            out = _final_layer(h, a0, a1, w1, bb1, w2, bb2, lin_W, lin_b,
                               batch_i32)
    return out


# speedup vs baseline: 2.2830x; 2.2830x over previous
import functools

import jax
import jax.numpy as jnp
from jax import lax
from jax.experimental import pallas as pl
from jax.experimental.pallas import tpu as pltpu
from jax.experimental.pallas import tpu_sc as plsc

NC = 2
NS = 16
NW = NC * NS

B_SEG = 64


def _acc_pad(n_rows):
    rpt = ((-(-(n_rows + 1) // NS)) + 7) // 8 * 8
    return rpt * NS, rpt


def _make_sc_agg(n_rows, d, nchunk, chunk):
    CHUNK = chunk
    npad, rpt = _acc_pad(n_rows)
    full, rem = divmod(rpt, CHUNK)

    mesh = plsc.VectorSubcoreMesh(
        core_axis_name="c", subcore_axis_name="s",
        num_cores=NC, num_subcores=NS)

    @functools.partial(
        pl.kernel,
        out_type=jax.ShapeDtypeStruct((NC, npad, d), jnp.float32),
        mesh=mesh,
        scratch_types=[
            pltpu.VMEM((nchunk, CHUNK), jnp.int32),
            pltpu.VMEM((nchunk, CHUNK), jnp.int32),
            pltpu.VMEM((CHUNK, d), jnp.float32),
            pltpu.VMEM((CHUNK, d), jnp.float32),
            pltpu.VMEM_SHARED((npad, d), jnp.float32),
            pltpu.SemaphoreType.DMA,
            pltpu.SemaphoreType.DMA,
        ],
        compiler_params=pltpu.CompilerParams(use_tc_tiling_on_sc=False),
    )
    def sc_agg(h_hbm, src_hbm, dst_hbm, out_hbm, idx_s, idx_d,
               rows0, rows1, acc, sem0, sem1):
        cid = lax.axis_index("c")
        sid = lax.axis_index("s")
        wid = sid * NC + cid

        def zero_body(i, _):
            r = i // (d // 16)
            col = (i % (d // 16)) * 16
            rows0[r, pl.ds(col, 16)] = jnp.zeros((16,), jnp.float32)
            return 0
        lax.fori_loop(0, CHUNK * d // 16, zero_body, 0)
        zbase = sid * rpt
        for k in range(full):
            pltpu.sync_copy(rows0, acc.at[pl.ds(zbase + k * CHUNK, CHUNK)])
        if rem:
            pltpu.sync_copy(rows0.at[pl.ds(0, rem)],
                            acc.at[pl.ds(zbase + full * CHUNK, rem)])
        plsc.subcore_barrier()

        pltpu.sync_copy(src_hbm.at[wid], idx_s)
        pltpu.sync_copy(dst_hbm.at[wid], idx_d)

        npairs = nchunk // 2
        pltpu.async_copy(h_hbm.at[idx_s.at[0]], rows0, sem0)

        def pair_body(p, _):
            c0 = 2 * p
            pltpu.async_copy(h_hbm.at[idx_s.at[c0 + 1]], rows1, sem1)
            pltpu.make_async_copy(h_hbm.at[idx_s.at[c0]], rows0, sem0).wait()
            pltpu.sync_copy(rows0, acc.at[idx_d.at[c0]], add=True)

            @pl.when(p + 1 < npairs)
            def _():
                pltpu.async_copy(h_hbm.at[idx_s.at[c0 + 2]], rows0, sem0)
            pltpu.make_async_copy(h_hbm.at[idx_s.at[c0 + 1]], rows1, sem1).wait()
            pltpu.sync_copy(rows1, acc.at[idx_d.at[c0 + 1]], add=True)
            return 0
        lax.fori_loop(0, npairs, pair_body, 0)
        plsc.subcore_barrier()

        pltpu.sync_copy(acc.at[pl.ds(zbase, rpt)],
                        out_hbm.at[cid, pl.ds(zbase, rpt)])

    return sc_agg


def _pick_bm(n):
    for bm in (512, 400, 256, 200, 128, 80, 40, 16, 8):
        if n % bm == 0:
            return bm
    return n


def _mlp_body(h_ref, a0_ref, a1_ref, w1_ref, b1_ref, w2_ref, b2_ref, o_ref):
    hs = h_ref[...] + a0_ref[...] + a1_ref[...]
    z = jnp.dot(hs, w1_ref[...], preferred_element_type=jnp.float32) + b1_ref[...]
    z = jnp.maximum(z, 0.0)
    z = jnp.dot(z, w2_ref[...], preferred_element_type=jnp.float32) + b2_ref[...]
    o_ref[...] = jnp.maximum(z, 0.0)


def _mlp_layer(h, a0, a1, w1, b1, w2, b2):
    n, din = h.shape
    hdim = w1.shape[1]
    bm = _pick_bm(n)
    return pl.pallas_call(
        _mlp_body,
        grid=(n // bm,),
        in_specs=[
            pl.BlockSpec((bm, din), lambda i: (i, 0)),
            pl.BlockSpec((bm, din), lambda i: (i, 0)),
            pl.BlockSpec((bm, din), lambda i: (i, 0)),
            pl.BlockSpec((din, hdim), lambda i: (0, 0)),
            pl.BlockSpec((1, hdim), lambda i: (0, 0)),
            pl.BlockSpec((hdim, hdim), lambda i: (0, 0)),
            pl.BlockSpec((1, hdim), lambda i: (0, 0)),
        ],
        out_specs=pl.BlockSpec((bm, hdim), lambda i: (i, 0)),
        out_shape=jax.ShapeDtypeStruct((n, hdim), jnp.float32),
    )(h, a0, a1, w1, b1.reshape(1, -1), w2, b2.reshape(1, -1))


def _final_layer(h, a0, a1, w1, b1, w2, b2, lin_w, lin_b, batch_i32):
    n, din = h.shape
    hdim = w1.shape[1]
    bm = _pick_bm(n)

    def body(h_ref, a0_ref, a1_ref, w1_ref, b1_ref, w2_ref, b2_ref,
             lw_ref, lb_ref, bt_ref, o_ref):
        hs = h_ref[...] + a0_ref[...] + a1_ref[...]
        z = jnp.dot(hs, w1_ref[...], preferred_element_type=jnp.float32) + b1_ref[...]
        z = jnp.maximum(z, 0.0)
        z = jnp.dot(z, w2_ref[...], preferred_element_type=jnp.float32) + b2_ref[...]
        hh = jnp.maximum(z, 0.0)
        y = jnp.dot(hh, lw_ref[...], preferred_element_type=jnp.float32)
        seg = lax.broadcasted_iota(jnp.int32, (bm, B_SEG), 1)
        oh = (bt_ref[...] == seg).astype(jnp.float32)
        contrib = jnp.sum(oh * y, axis=0, keepdims=True)

        @pl.when(pl.program_id(0) == 0)
        def _():
            o_ref[...] = contrib + lb_ref[...]

        @pl.when(pl.program_id(0) != 0)
        def _():
            o_ref[...] = o_ref[...] + contrib

    out = pl.pallas_call(
        body,
        grid=(n // bm,),
        in_specs=[
            pl.BlockSpec((bm, din), lambda i: (i, 0)),
            pl.BlockSpec((bm, din), lambda i: (i, 0)),
            pl.BlockSpec((bm, din), lambda i: (i, 0)),
            pl.BlockSpec((din, hdim), lambda i: (0, 0)),
            pl.BlockSpec((1, hdim), lambda i: (0, 0)),
            pl.BlockSpec((hdim, hdim), lambda i: (0, 0)),
            pl.BlockSpec((1, hdim), lambda i: (0, 0)),
            pl.BlockSpec((hdim, 1), lambda i: (0, 0)),
            pl.BlockSpec((1, 1), lambda i: (0, 0)),
            pl.BlockSpec((bm, 1), lambda i: (i, 0)),
        ],
        out_specs=pl.BlockSpec((1, B_SEG), lambda i: (0, 0)),
        out_shape=jax.ShapeDtypeStruct((1, B_SEG), jnp.float32),
    )(h, a0, a1, w1, b1.reshape(1, -1), w2, b2.reshape(1, -1),
      lin_w, lin_b.reshape(1, 1), batch_i32)
    return out[0]


def kernel(x, edge_index, edge_attr, batch,
           W1_0, b1_0, W2_0, b2_0, W1_1, b1_1, W2_1, b2_1,
           W1_2, b1_2, W2_2, b2_2, lin_W, lin_b):
    n, d = x.shape
    e = edge_index.shape[1]
    npad, _ = _acc_pad(n)

    def pad_edges(chunk):
        nchunk = -(-e // (NW * chunk))
        nchunk += nchunk % 2
        e_pad = NW * nchunk * chunk
        src = edge_index[0]
        dst = edge_index[1]
        if e_pad > e:
            fill = jnp.arange(e_pad - e, dtype=jnp.int32)
            src = jnp.concatenate([src, fill % n])
            dst = jnp.concatenate([dst, n + (fill % (npad - n))])
        return src.reshape(NW, nchunk, chunk), dst.reshape(NW, nchunk, chunk), nchunk

    edge_plans = {128: pad_edges(128), 96: pad_edges(96)}

    batch_i32 = batch.astype(jnp.int32).reshape(n, 1)

    layers = [(W1_0, b1_0, W2_0, b2_0), (W1_1, b1_1, W2_1, b2_1),
              (W1_2, b1_2, W2_2, b2_2)]

    h = x
    for li, (w1, bb1, w2, bb2) in enumerate(layers):
        chunk = 96 if h.shape[1] > 64 else 128
        src, dst, nchunk = edge_plans[chunk]
        sc_agg = _make_sc_agg(n, h.shape[1], nchunk, chunk)
        agg = sc_agg(h, src, dst)
        a0 = agg[0, :n]
        a1 = agg[1, :n]
        if li < 2:
            h = _mlp_layer(h, a0, a1, w1, bb1, w2, bb2)
        else:
            out = _final_layer(h, a0, a1, w1, bb1, w2, bb2, lin_W, lin_b,
                               batch_i32)
    return out


# speedup vs baseline: 2.5901x; 1.1345x over previous
import functools

import jax
import jax.numpy as jnp
from jax import lax
from jax.experimental import pallas as pl
from jax.experimental.pallas import tpu as pltpu
from jax.experimental.pallas import tpu_sc as plsc

NC = 2
NS = 16
NW = NC * NS

B_SEG = 64


def _acc_pad(n_rows):
    rpt = ((-(-(n_rows + 1) // NS)) + 7) // 8 * 8
    return rpt * NS, rpt


def _make_sc_agg(n_rows, d, nchunk, chunk):
    CHUNK = chunk
    npad, rpt = _acc_pad(n_rows)
    full, rem = divmod(rpt, CHUNK)

    mesh = plsc.VectorSubcoreMesh(
        core_axis_name="c", subcore_axis_name="s",
        num_cores=NC, num_subcores=NS)

    @functools.partial(
        pl.kernel,
        out_type=jax.ShapeDtypeStruct((NC, npad, d), jnp.float32),
        mesh=mesh,
        scratch_types=[
            pltpu.VMEM((nchunk, CHUNK), jnp.int32),
            pltpu.VMEM((nchunk, CHUNK), jnp.int32),
            pltpu.VMEM((CHUNK, d), jnp.float32),
            pltpu.VMEM((CHUNK, d), jnp.float32),
            pltpu.VMEM_SHARED((npad, d), jnp.float32),
            pltpu.SemaphoreType.DMA,
            pltpu.SemaphoreType.DMA,
        ],
        compiler_params=pltpu.CompilerParams(use_tc_tiling_on_sc=False),
    )
    def sc_agg(h_hbm, src_hbm, dst_hbm, out_hbm, idx_s, idx_d,
               rows0, rows1, acc, sem0, sem1):
        cid = lax.axis_index("c")
        sid = lax.axis_index("s")
        wid = sid * NC + cid

        def zero_body(i, _):
            r = i // (d // 16)
            col = (i % (d // 16)) * 16
            rows0[r, pl.ds(col, 16)] = jnp.zeros((16,), jnp.float32)
            return 0
        lax.fori_loop(0, CHUNK * d // 16, zero_body, 0)
        zbase = sid * rpt
        for k in range(full):
            pltpu.sync_copy(rows0, acc.at[pl.ds(zbase + k * CHUNK, CHUNK)])
        if rem:
            pltpu.sync_copy(rows0.at[pl.ds(0, rem)],
                            acc.at[pl.ds(zbase + full * CHUNK, rem)])
        plsc.subcore_barrier()

        pltpu.sync_copy(src_hbm.at[wid], idx_s)
        pltpu.sync_copy(dst_hbm.at[wid], idx_d)

        npairs = nchunk // 2
        pltpu.async_copy(h_hbm.at[idx_s.at[0]], rows0, sem0)

        def pair_body(p, _):
            c0 = 2 * p
            pltpu.async_copy(h_hbm.at[idx_s.at[c0 + 1]], rows1, sem1)
            pltpu.make_async_copy(h_hbm.at[idx_s.at[c0]], rows0, sem0).wait()
            pltpu.sync_copy(rows0, acc.at[idx_d.at[c0]], add=True)

            @pl.when(p + 1 < npairs)
            def _():
                pltpu.async_copy(h_hbm.at[idx_s.at[c0 + 2]], rows0, sem0)
            pltpu.make_async_copy(h_hbm.at[idx_s.at[c0 + 1]], rows1, sem1).wait()
            pltpu.sync_copy(rows1, acc.at[idx_d.at[c0 + 1]], add=True)
            return 0
        lax.fori_loop(0, npairs, pair_body, 0)
        plsc.subcore_barrier()

        pltpu.sync_copy(acc.at[pl.ds(zbase, rpt)],
                        out_hbm.at[cid, pl.ds(zbase, rpt)])

    return sc_agg


def _pick_bm(n):
    for bm in (1000, 512, 400, 256, 200, 128, 80, 40, 16, 8):
        if n % bm == 0:
            return bm
    return n


def _mlp_body(h_ref, agg_ref, w1_ref, b1_ref, w2_ref, b2_ref, o_ref):
    hs = h_ref[...] + agg_ref[0] + agg_ref[1]
    z = jnp.dot(hs, w1_ref[...], preferred_element_type=jnp.float32) + b1_ref[...]
    z = jnp.maximum(z, 0.0)
    z = jnp.dot(z, w2_ref[...], preferred_element_type=jnp.float32) + b2_ref[...]
    o_ref[...] = jnp.maximum(z, 0.0)


def _mlp_layer(h, agg, w1, b1, w2, b2):
    n, din = h.shape
    hdim = w1.shape[1]
    bm = _pick_bm(n)
    return pl.pallas_call(
        _mlp_body,
        grid=(n // bm,),
        in_specs=[
            pl.BlockSpec((bm, din), lambda i: (i, 0)),
            pl.BlockSpec((2, bm, din), lambda i: (0, i, 0)),
            pl.BlockSpec((din, hdim), lambda i: (0, 0)),
            pl.BlockSpec((1, hdim), lambda i: (0, 0)),
            pl.BlockSpec((hdim, hdim), lambda i: (0, 0)),
            pl.BlockSpec((1, hdim), lambda i: (0, 0)),
        ],
        out_specs=pl.BlockSpec((bm, hdim), lambda i: (i, 0)),
        out_shape=jax.ShapeDtypeStruct((n, hdim), jnp.float32),
    )(h, agg, w1, b1.reshape(1, -1), w2, b2.reshape(1, -1))


def _final_layer(h, agg, w1, b1, w2, b2, lin_w, lin_b, batch_i32):
    n, din = h.shape
    hdim = w1.shape[1]
    bm = _pick_bm(n)

    def body(h_ref, agg_ref, w1_ref, b1_ref, w2_ref, b2_ref,
             lw_ref, lb_ref, bt_ref, o_ref):
        hs = h_ref[...] + agg_ref[0] + agg_ref[1]
        z = jnp.dot(hs, w1_ref[...], preferred_element_type=jnp.float32) + b1_ref[...]
        z = jnp.maximum(z, 0.0)
        z = jnp.dot(z, w2_ref[...], preferred_element_type=jnp.float32) + b2_ref[...]
        hh = jnp.maximum(z, 0.0)
        y = jnp.dot(hh, lw_ref[...], preferred_element_type=jnp.float32)
        seg = lax.broadcasted_iota(jnp.int32, (bm, B_SEG), 1)
        oh = (bt_ref[...] == seg).astype(jnp.float32)
        contrib = jnp.sum(oh * y, axis=0, keepdims=True)

        @pl.when(pl.program_id(0) == 0)
        def _():
            o_ref[...] = contrib + lb_ref[...]

        @pl.when(pl.program_id(0) != 0)
        def _():
            o_ref[...] = o_ref[...] + contrib

    out = pl.pallas_call(
        body,
        grid=(n // bm,),
        in_specs=[
            pl.BlockSpec((bm, din), lambda i: (i, 0)),
            pl.BlockSpec((2, bm, din), lambda i: (0, i, 0)),
            pl.BlockSpec((din, hdim), lambda i: (0, 0)),
            pl.BlockSpec((1, hdim), lambda i: (0, 0)),
            pl.BlockSpec((hdim, hdim), lambda i: (0, 0)),
            pl.BlockSpec((1, hdim), lambda i: (0, 0)),
            pl.BlockSpec((hdim, 1), lambda i: (0, 0)),
            pl.BlockSpec((1, 1), lambda i: (0, 0)),
            pl.BlockSpec((bm, 1), lambda i: (i, 0)),
        ],
        out_specs=pl.BlockSpec((1, B_SEG), lambda i: (0, 0)),
        out_shape=jax.ShapeDtypeStruct((1, B_SEG), jnp.float32),
    )(h, agg, w1, b1.reshape(1, -1), w2, b2.reshape(1, -1),
      lin_w, lin_b.reshape(1, 1), batch_i32)
    return out[0]


def kernel(x, edge_index, edge_attr, batch,
           W1_0, b1_0, W2_0, b2_0, W1_1, b1_1, W2_1, b2_1,
           W1_2, b1_2, W2_2, b2_2, lin_W, lin_b):
    n, d = x.shape
    e = edge_index.shape[1]
    npad, _ = _acc_pad(n)

    def pad_edges(chunk):
        nchunk = -(-e // (NW * chunk))
        nchunk += nchunk % 2
        e_pad = NW * nchunk * chunk
        src = edge_index[0]
        dst = edge_index[1]
        if e_pad > e:
            fill = jnp.arange(e_pad - e, dtype=jnp.int32)
            src = jnp.concatenate([src, fill % n])
            dst = jnp.concatenate([dst, n + (fill % (npad - n))])
        return src.reshape(NW, nchunk, chunk), dst.reshape(NW, nchunk, chunk), nchunk

    edge_plans = {128: pad_edges(128), 96: pad_edges(96)}

    batch_i32 = batch.astype(jnp.int32).reshape(n, 1)

    layers = [(W1_0, b1_0, W2_0, b2_0), (W1_1, b1_1, W2_1, b2_1),
              (W1_2, b1_2, W2_2, b2_2)]

    h = x
    for li, (w1, bb1, w2, bb2) in enumerate(layers):
        chunk = 96 if h.shape[1] > 64 else 128
        src, dst, nchunk = edge_plans[chunk]
        sc_agg = _make_sc_agg(n, h.shape[1], nchunk, chunk)
        agg = sc_agg(h, src, dst)
        if li < 2:
            h = _mlp_layer(h, agg, w1, bb1, w2, bb2)
        else:
            out = _final_layer(h, agg, w1, bb1, w2, bb2, lin_W, lin_b,
                               batch_i32)
    return out


# speedup vs baseline: 2.8580x; 1.1034x over previous
import functools

import jax
import jax.numpy as jnp
from jax import lax
from jax.experimental import pallas as pl
from jax.experimental.pallas import tpu as pltpu
from jax.experimental.pallas import tpu_sc as plsc

NC = 2
NS = 16
NW = NC * NS

B_SEG = 64


def _acc_pad(n_rows):
    rpt = ((-(-(n_rows + 1) // NS)) + 7) // 8 * 8
    return rpt * NS, rpt


def _make_sc_agg(n_rows, d, nchunk, chunk, nbuf):
    CHUNK = chunk
    npad, rpt = _acc_pad(n_rows)
    full, rem = divmod(rpt, CHUNK)

    mesh = plsc.VectorSubcoreMesh(
        core_axis_name="c", subcore_axis_name="s",
        num_cores=NC, num_subcores=NS)

    @functools.partial(
        pl.kernel,
        out_type=jax.ShapeDtypeStruct((NC, npad, d), jnp.float32),
        mesh=mesh,
        scratch_types=[
            pltpu.VMEM((nchunk, CHUNK), jnp.int32),
            pltpu.VMEM((nchunk, CHUNK), jnp.int32),
            [pltpu.VMEM((CHUNK, d), jnp.float32) for _ in range(nbuf)],
            pltpu.VMEM_SHARED((npad, d), jnp.float32),
            [pltpu.SemaphoreType.DMA for _ in range(nbuf)],
        ],
        compiler_params=pltpu.CompilerParams(use_tc_tiling_on_sc=False),
    )
    def sc_agg(h_hbm, src_hbm, dst_hbm, out_hbm, idx_s, idx_d,
               bufs, acc, sems):
        cid = lax.axis_index("c")
        sid = lax.axis_index("s")
        wid = sid * NC + cid

        pltpu.async_copy(src_hbm.at[wid], idx_s, sems[0])
        pltpu.async_copy(dst_hbm.at[wid], idx_d, sems[1])

        zbuf = bufs[nbuf - 1]

        def zero_body(i, _):
            r = i // (d // 16)
            col = (i % (d // 16)) * 16
            zbuf[r, pl.ds(col, 16)] = jnp.zeros((16,), jnp.float32)
            return 0
        lax.fori_loop(0, CHUNK * d // 16, zero_body, 0)
        zbase = sid * rpt
        for k in range(full):
            pltpu.sync_copy(zbuf, acc.at[pl.ds(zbase + k * CHUNK, CHUNK)])
        if rem:
            pltpu.sync_copy(zbuf.at[pl.ds(0, rem)],
                            acc.at[pl.ds(zbase + full * CHUNK, rem)])

        pltpu.make_async_copy(src_hbm.at[wid], idx_s, sems[0]).wait()
        pltpu.make_async_copy(dst_hbm.at[wid], idx_d, sems[1]).wait()

        for j in range(nbuf):
            pltpu.async_copy(h_hbm.at[idx_s.at[j]], bufs[j], sems[j])
        plsc.subcore_barrier()

        ngroups = nchunk // nbuf

        def group_body(g, _):
            base = nbuf * g
            for j in range(nbuf):
                c = base + j
                pltpu.make_async_copy(h_hbm.at[idx_s.at[c]], bufs[j],
                                      sems[j]).wait()
                pltpu.sync_copy(bufs[j], acc.at[idx_d.at[c]], add=True)

                @pl.when(c + nbuf < nchunk)
                def _():
                    pltpu.async_copy(h_hbm.at[idx_s.at[c + nbuf]], bufs[j],
                                     sems[j])
            return 0
        lax.fori_loop(0, ngroups, group_body, 0)
        plsc.subcore_barrier()

        pltpu.sync_copy(acc.at[pl.ds(zbase, rpt)],
                        out_hbm.at[cid, pl.ds(zbase, rpt)])

    return sc_agg


def _pick_bm(n):
    for bm in (1000, 512, 400, 256, 200, 128, 80, 40, 16, 8):
        if n % bm == 0:
            return bm
    return n


def _mlp_body(h_ref, agg_ref, w1_ref, b1_ref, w2_ref, b2_ref, o_ref):
    hs = h_ref[...] + agg_ref[0] + agg_ref[1]
    z = jnp.dot(hs, w1_ref[...], preferred_element_type=jnp.float32) + b1_ref[...]
    z = jnp.maximum(z, 0.0)
    z = jnp.dot(z, w2_ref[...], preferred_element_type=jnp.float32) + b2_ref[...]
    o_ref[...] = jnp.maximum(z, 0.0)


def _mlp_layer(h, agg, w1, b1, w2, b2):
    n, din = h.shape
    hdim = w1.shape[1]
    bm = _pick_bm(n)
    return pl.pallas_call(
        _mlp_body,
        grid=(n // bm,),
        in_specs=[
            pl.BlockSpec((bm, din), lambda i: (i, 0)),
            pl.BlockSpec((2, bm, din), lambda i: (0, i, 0)),
            pl.BlockSpec((din, hdim), lambda i: (0, 0)),
            pl.BlockSpec((1, hdim), lambda i: (0, 0)),
            pl.BlockSpec((hdim, hdim), lambda i: (0, 0)),
            pl.BlockSpec((1, hdim), lambda i: (0, 0)),
        ],
        out_specs=pl.BlockSpec((bm, hdim), lambda i: (i, 0)),
        out_shape=jax.ShapeDtypeStruct((n, hdim), jnp.float32),
    )(h, agg, w1, b1.reshape(1, -1), w2, b2.reshape(1, -1))


def _final_layer(h, agg, w1, b1, w2, b2, lin_w, lin_b, batch_i32):
    n, din = h.shape
    hdim = w1.shape[1]
    bm = _pick_bm(n)

    def body(h_ref, agg_ref, w1_ref, b1_ref, w2_ref, b2_ref,
             lw_ref, lb_ref, bt_ref, o_ref):
        hs = h_ref[...] + agg_ref[0] + agg_ref[1]
        z = jnp.dot(hs, w1_ref[...], preferred_element_type=jnp.float32) + b1_ref[...]
        z = jnp.maximum(z, 0.0)
        z = jnp.dot(z, w2_ref[...], preferred_element_type=jnp.float32) + b2_ref[...]
        hh = jnp.maximum(z, 0.0)
        y = jnp.dot(hh, lw_ref[...], preferred_element_type=jnp.float32)
        seg = lax.broadcasted_iota(jnp.int32, (bm, B_SEG), 1)
        oh = (bt_ref[...] == seg).astype(jnp.float32)
        contrib = jnp.sum(oh * y, axis=0, keepdims=True)

        @pl.when(pl.program_id(0) == 0)
        def _():
            o_ref[...] = contrib + lb_ref[...]

        @pl.when(pl.program_id(0) != 0)
        def _():
            o_ref[...] = o_ref[...] + contrib

    out = pl.pallas_call(
        body,
        grid=(n // bm,),
        in_specs=[
            pl.BlockSpec((bm, din), lambda i: (i, 0)),
            pl.BlockSpec((2, bm, din), lambda i: (0, i, 0)),
            pl.BlockSpec((din, hdim), lambda i: (0, 0)),
            pl.BlockSpec((1, hdim), lambda i: (0, 0)),
            pl.BlockSpec((hdim, hdim), lambda i: (0, 0)),
            pl.BlockSpec((1, hdim), lambda i: (0, 0)),
            pl.BlockSpec((hdim, 1), lambda i: (0, 0)),
            pl.BlockSpec((1, 1), lambda i: (0, 0)),
            pl.BlockSpec((bm, 1), lambda i: (i, 0)),
        ],
        out_specs=pl.BlockSpec((1, B_SEG), lambda i: (0, 0)),
        out_shape=jax.ShapeDtypeStruct((1, B_SEG), jnp.float32),
    )(h, agg, w1, b1.reshape(1, -1), w2, b2.reshape(1, -1),
      lin_w, lin_b.reshape(1, 1), batch_i32)
    return out[0]


def kernel(x, edge_index, edge_attr, batch,
           W1_0, b1_0, W2_0, b2_0, W1_1, b1_1, W2_1, b2_1,
           W1_2, b1_2, W2_2, b2_2, lin_W, lin_b):
    n, d = x.shape
    e = edge_index.shape[1]
    npad, _ = _acc_pad(n)

    def pad_edges(chunk, nbuf):
        nchunk = -(-e // (NW * chunk))
        nchunk = -(-nchunk // nbuf) * nbuf
        e_pad = NW * nchunk * chunk
        src = edge_index[0]
        dst = edge_index[1]
        if e_pad > e:
            fill = jnp.arange(e_pad - e, dtype=jnp.int32)
            src = jnp.concatenate([src, fill % n])
            dst = jnp.concatenate([dst, n + (fill % (npad - n))])
        return src.reshape(NW, nchunk, chunk), dst.reshape(NW, nchunk, chunk), nchunk

    plans = {128: (96, 2), 64: (128, 3)}
    edge_plans = {dd: (cfg[0], cfg[1]) + pad_edges(*cfg)
                  for dd, cfg in plans.items()}

    batch_i32 = batch.astype(jnp.int32).reshape(n, 1)

    layers = [(W1_0, b1_0, W2_0, b2_0), (W1_1, b1_1, W2_1, b2_1),
              (W1_2, b1_2, W2_2, b2_2)]

    h = x
    for li, (w1, bb1, w2, bb2) in enumerate(layers):
        chunk, nbuf, src_p, dst_p, nchunk = edge_plans[min(h.shape[1], 128)]
        sc_agg = _make_sc_agg(n, h.shape[1], nchunk, chunk, nbuf)
        agg = sc_agg(h, src_p, dst_p)
        if li < 2:
            h = _mlp_layer(h, agg, w1, bb1, w2, bb2)
        else:
            out = _final_layer(h, agg, w1, bb1, w2, bb2, lin_W, lin_b,
                               batch_i32)
    return out


# speedup vs baseline: 2.9527x; 1.0332x over previous
import functools

import jax
import jax.numpy as jnp
from jax import lax
from jax.experimental import pallas as pl
from jax.experimental.pallas import tpu as pltpu
from jax.experimental.pallas import tpu_sc as plsc

NC = 2
NS = 16
NW = NC * NS

B_SEG = 64


def _acc_pad(n_rows):
    rpt = ((-(-(n_rows + 1) // NS)) + 7) // 8 * 8
    return rpt * NS, rpt


def _make_sc_agg(n_rows, d, nchunk, chunk, nbuf):
    CHUNK = chunk
    npad, rpt = _acc_pad(n_rows)
    full, rem = divmod(rpt, CHUNK)

    mesh = plsc.VectorSubcoreMesh(
        core_axis_name="c", subcore_axis_name="s",
        num_cores=NC, num_subcores=NS)

    @functools.partial(
        pl.kernel,
        out_type=jax.ShapeDtypeStruct((NC, npad, d), jnp.float32),
        mesh=mesh,
        scratch_types=[
            pltpu.VMEM((nchunk, CHUNK), jnp.int32),
            pltpu.VMEM((nchunk, CHUNK), jnp.int32),
            [pltpu.VMEM((CHUNK, d), jnp.float32) for _ in range(nbuf)],
            pltpu.VMEM_SHARED((npad, d), jnp.float32),
            [pltpu.SemaphoreType.DMA for _ in range(nbuf)],
        ],
        compiler_params=pltpu.CompilerParams(use_tc_tiling_on_sc=False),
    )
    def sc_agg(h_hbm, src_hbm, dst_hbm, out_hbm, idx_s, idx_d,
               bufs, acc, sems):
        cid = lax.axis_index("c")
        sid = lax.axis_index("s")
        wid = sid * NC + cid

        pltpu.async_copy(src_hbm.at[wid], idx_s, sems[0])
        pltpu.async_copy(dst_hbm.at[wid], idx_d, sems[1])

        zbuf = bufs[nbuf - 1]

        def zero_body(i, _):
            r = i // (d // 16)
            col = (i % (d // 16)) * 16
            zbuf[r, pl.ds(col, 16)] = jnp.zeros((16,), jnp.float32)
            return 0
        lax.fori_loop(0, CHUNK * d // 16, zero_body, 0)
        zbase = sid * rpt
        for k in range(full):
            pltpu.sync_copy(zbuf, acc.at[pl.ds(zbase + k * CHUNK, CHUNK)])
        if rem:
            pltpu.sync_copy(zbuf.at[pl.ds(0, rem)],
                            acc.at[pl.ds(zbase + full * CHUNK, rem)])

        pltpu.make_async_copy(src_hbm.at[wid], idx_s, sems[0]).wait()
        pltpu.make_async_copy(dst_hbm.at[wid], idx_d, sems[1]).wait()

        for j in range(nbuf):
            pltpu.async_copy(h_hbm.at[idx_s.at[j]], bufs[j], sems[j])
        plsc.subcore_barrier()

        ngroups = nchunk // nbuf

        def group_body(g, _):
            base = nbuf * g
            for j in range(nbuf):
                c = base + j
                pltpu.make_async_copy(h_hbm.at[idx_s.at[c]], bufs[j],
                                      sems[j]).wait()
                pltpu.sync_copy(bufs[j], acc.at[idx_d.at[c]], add=True)

                @pl.when(c + nbuf < nchunk)
                def _():
                    pltpu.async_copy(h_hbm.at[idx_s.at[c + nbuf]], bufs[j],
                                     sems[j])
            return 0
        lax.fori_loop(0, ngroups, group_body, 0)
        plsc.subcore_barrier()

        pltpu.sync_copy(acc.at[pl.ds(zbase, rpt)],
                        out_hbm.at[cid, pl.ds(zbase, rpt)])

    return sc_agg


def _pick_bm(n):
    for bm in (1000, 512, 400, 256, 200, 128, 80, 40, 16, 8):
        if n % bm == 0:
            return bm
    return n


def _mlp_body(h_ref, agg_ref, w1_ref, b1_ref, w2_ref, b2_ref, o_ref):
    hs = h_ref[...] + agg_ref[0] + agg_ref[1]
    z = jnp.dot(hs, w1_ref[...], preferred_element_type=jnp.float32) + b1_ref[...]
    z = jnp.maximum(z, 0.0)
    z = jnp.dot(z, w2_ref[...], preferred_element_type=jnp.float32) + b2_ref[...]
    o_ref[...] = jnp.maximum(z, 0.0)


def _mlp_layer(h, agg, w1, b1, w2, b2):
    n, din = h.shape
    hdim = w1.shape[1]
    bm = _pick_bm(n)
    return pl.pallas_call(
        _mlp_body,
        grid=(n // bm,),
        in_specs=[
            pl.BlockSpec((bm, din), lambda i: (i, 0)),
            pl.BlockSpec((2, bm, din), lambda i: (0, i, 0)),
            pl.BlockSpec((din, hdim), lambda i: (0, 0)),
            pl.BlockSpec((1, hdim), lambda i: (0, 0)),
            pl.BlockSpec((hdim, hdim), lambda i: (0, 0)),
            pl.BlockSpec((1, hdim), lambda i: (0, 0)),
        ],
        out_specs=pl.BlockSpec((bm, hdim), lambda i: (i, 0)),
        out_shape=jax.ShapeDtypeStruct((n, hdim), jnp.float32),
    )(h, agg, w1, b1.reshape(1, -1), w2, b2.reshape(1, -1))


def _final_layer(h, agg, w1, b1, w2, b2, lin_w, lin_b, batch_i32):
    n, din = h.shape
    hdim = w1.shape[1]
    bm = _pick_bm(n)

    def body(h_ref, agg_ref, w1_ref, b1_ref, w2_ref, b2_ref,
             lw_ref, lb_ref, bt_ref, o_ref):
        hs = h_ref[...] + agg_ref[0] + agg_ref[1]
        z = jnp.dot(hs, w1_ref[...], preferred_element_type=jnp.float32) + b1_ref[...]
        z = jnp.maximum(z, 0.0)
        z = jnp.dot(z, w2_ref[...], preferred_element_type=jnp.float32) + b2_ref[...]
        hh = jnp.maximum(z, 0.0)
        y = jnp.dot(hh, lw_ref[...], preferred_element_type=jnp.float32)
        seg = lax.broadcasted_iota(jnp.int32, (bm, B_SEG), 1)
        oh = (bt_ref[...] == seg).astype(jnp.float32)
        contrib = jnp.sum(oh * y, axis=0, keepdims=True)

        @pl.when(pl.program_id(0) == 0)
        def _():
            o_ref[...] = contrib + lb_ref[...]

        @pl.when(pl.program_id(0) != 0)
        def _():
            o_ref[...] = o_ref[...] + contrib

    out = pl.pallas_call(
        body,
        grid=(n // bm,),
        in_specs=[
            pl.BlockSpec((bm, din), lambda i: (i, 0)),
            pl.BlockSpec((2, bm, din), lambda i: (0, i, 0)),
            pl.BlockSpec((din, hdim), lambda i: (0, 0)),
            pl.BlockSpec((1, hdim), lambda i: (0, 0)),
            pl.BlockSpec((hdim, hdim), lambda i: (0, 0)),
            pl.BlockSpec((1, hdim), lambda i: (0, 0)),
            pl.BlockSpec((hdim, 1), lambda i: (0, 0)),
            pl.BlockSpec((1, 1), lambda i: (0, 0)),
            pl.BlockSpec((bm, 1), lambda i: (i, 0)),
        ],
        out_specs=pl.BlockSpec((1, B_SEG), lambda i: (0, 0)),
        out_shape=jax.ShapeDtypeStruct((1, B_SEG), jnp.float32),
    )(h, agg, w1, b1.reshape(1, -1), w2, b2.reshape(1, -1),
      lin_w, lin_b.reshape(1, 1), batch_i32)
    return out[0]


def kernel(x, edge_index, edge_attr, batch,
           W1_0, b1_0, W2_0, b2_0, W1_1, b1_1, W2_1, b2_1,
           W1_2, b1_2, W2_2, b2_2, lin_W, lin_b):
    n, d = x.shape
    e = edge_index.shape[1]
    npad, _ = _acc_pad(n)

    def pad_edges(chunk, nbuf):
        nchunk = -(-e // (NW * chunk))
        nchunk = -(-nchunk // nbuf) * nbuf
        e_pad = NW * nchunk * chunk
        src = edge_index[0]
        dst = edge_index[1]
        if e_pad > e:
            fill = jnp.arange(e_pad - e, dtype=jnp.int32)
            src = jnp.concatenate([src, fill % n])
            dst = jnp.concatenate([dst, n + (fill % (npad - n))])
        return src.reshape(NW, nchunk, chunk), dst.reshape(NW, nchunk, chunk), nchunk

    plans = {128: (112, 2), 64: (128, 4)}
    edge_plans = {dd: (cfg[0], cfg[1]) + pad_edges(*cfg)
                  for dd, cfg in plans.items()}

    batch_i32 = batch.astype(jnp.int32).reshape(n, 1)

    layers = [(W1_0, b1_0, W2_0, b2_0), (W1_1, b1_1, W2_1, b2_1),
              (W1_2, b1_2, W2_2, b2_2)]

    h = x
    for li, (w1, bb1, w2, bb2) in enumerate(layers):
        chunk, nbuf, src_p, dst_p, nchunk = edge_plans[min(h.shape[1], 128)]
        sc_agg = _make_sc_agg(n, h.shape[1], nchunk, chunk, nbuf)
        agg = sc_agg(h, src_p, dst_p)
        if li < 2:
            h = _mlp_layer(h, agg, w1, bb1, w2, bb2)
        else:
            out = _final_layer(h, agg, w1, bb1, w2, bb2, lin_W, lin_b,
                               batch_i32)
    return out


# speedup vs baseline: 3.0545x; 1.0345x over previous
import functools

import jax
import jax.numpy as jnp
from jax import lax
from jax.experimental import pallas as pl
from jax.experimental.pallas import tpu as pltpu
from jax.experimental.pallas import tpu_sc as plsc

NC = 2
NS = 16
NW = NC * NS

B_SEG = 64


def _acc_pad(n_rows):
    rpt = ((-(-(n_rows + 1) // NS)) + 7) // 8 * 8
    return rpt * NS, rpt


def _make_sc_agg(n_rows, d, nchunk, chunk, nbuf):
    CHUNK = chunk
    npad, rpt = _acc_pad(n_rows)
    full, rem = divmod(rpt, CHUNK)

    mesh = plsc.VectorSubcoreMesh(
        core_axis_name="c", subcore_axis_name="s",
        num_cores=NC, num_subcores=NS)

    @functools.partial(
        pl.kernel,
        out_type=jax.ShapeDtypeStruct((NC, npad, d), jnp.float32),
        mesh=mesh,
        scratch_types=[
            pltpu.VMEM((nchunk, CHUNK), jnp.int32),
            pltpu.VMEM((nchunk, CHUNK), jnp.int32),
            [pltpu.VMEM((CHUNK, d), jnp.float32) for _ in range(nbuf)],
            pltpu.VMEM_SHARED((npad, d), jnp.float32),
            [pltpu.SemaphoreType.DMA for _ in range(nbuf)],
        ],
        compiler_params=pltpu.CompilerParams(use_tc_tiling_on_sc=False),
    )
    def sc_agg(h_hbm, src_hbm, dst_hbm, out_hbm, idx_s, idx_d,
               bufs, acc, sems):
        cid = lax.axis_index("c")
        sid = lax.axis_index("s")
        wid = sid * NC + cid

        pltpu.async_copy(src_hbm.at[wid], idx_s, sems[0])
        pltpu.async_copy(dst_hbm.at[wid], idx_d, sems[1])

        zbuf = bufs[nbuf - 1]

        def zero_body(i, _):
            r = i // (d // 16)
            col = (i % (d // 16)) * 16
            zbuf[r, pl.ds(col, 16)] = jnp.zeros((16,), jnp.float32)
            return 0
        lax.fori_loop(0, CHUNK * d // 16, zero_body, 0)
        zbase = sid * rpt
        for k in range(full):
            pltpu.sync_copy(zbuf, acc.at[pl.ds(zbase + k * CHUNK, CHUNK)])
        if rem:
            pltpu.sync_copy(zbuf.at[pl.ds(0, rem)],
                            acc.at[pl.ds(zbase + full * CHUNK, rem)])

        pltpu.make_async_copy(src_hbm.at[wid], idx_s, sems[0]).wait()
        pltpu.make_async_copy(dst_hbm.at[wid], idx_d, sems[1]).wait()

        for j in range(nbuf):
            pltpu.async_copy(h_hbm.at[idx_s.at[j]], bufs[j], sems[j])
        plsc.subcore_barrier()

        ngroups = nchunk // nbuf

        def group_body(g, _):
            base = nbuf * g
            for j in range(nbuf):
                c = base + j
                pltpu.make_async_copy(h_hbm.at[idx_s.at[c]], bufs[j],
                                      sems[j]).wait()
                pltpu.sync_copy(bufs[j], acc.at[idx_d.at[c]], add=True)

                @pl.when(c + nbuf < nchunk)
                def _():
                    pltpu.async_copy(h_hbm.at[idx_s.at[c + nbuf]], bufs[j],
                                     sems[j])
            return 0
        lax.fori_loop(0, ngroups, group_body, 0)
        plsc.subcore_barrier()

        pltpu.sync_copy(acc.at[pl.ds(zbase, rpt)],
                        out_hbm.at[cid, pl.ds(zbase, rpt)])

    return sc_agg


def _pick_bm(n):
    for bm in (2000, 1000, 512, 400, 256, 200, 128, 80, 40, 16, 8):
        if n % bm == 0:
            return bm
    return n


def _mlp_body(h_ref, agg_ref, w1_ref, b1_ref, w2_ref, b2_ref, o_ref):
    hs = h_ref[...] + agg_ref[0] + agg_ref[1]
    z = jnp.dot(hs, w1_ref[...], preferred_element_type=jnp.float32) + b1_ref[...]
    z = jnp.maximum(z, 0.0)
    z = jnp.dot(z, w2_ref[...], preferred_element_type=jnp.float32) + b2_ref[...]
    o_ref[...] = jnp.maximum(z, 0.0)


def _mlp_layer(h, agg, w1, b1, w2, b2):
    n, din = h.shape
    hdim = w1.shape[1]
    bm = _pick_bm(n)
    return pl.pallas_call(
        _mlp_body,
        grid=(n // bm,),
        in_specs=[
            pl.BlockSpec((bm, din), lambda i: (i, 0)),
            pl.BlockSpec((2, bm, din), lambda i: (0, i, 0)),
            pl.BlockSpec((din, hdim), lambda i: (0, 0)),
            pl.BlockSpec((1, hdim), lambda i: (0, 0)),
            pl.BlockSpec((hdim, hdim), lambda i: (0, 0)),
            pl.BlockSpec((1, hdim), lambda i: (0, 0)),
        ],
        out_specs=pl.BlockSpec((bm, hdim), lambda i: (i, 0)),
        out_shape=jax.ShapeDtypeStruct((n, hdim), jnp.float32),
    )(h, agg, w1, b1.reshape(1, -1), w2, b2.reshape(1, -1))


def _final_layer(h, agg, w1, b1, w2, b2, lin_w, lin_b, batch_i32):
    n, din = h.shape
    hdim = w1.shape[1]
    bm = _pick_bm(n)

    def body(h_ref, agg_ref, w1_ref, b1_ref, w2_ref, b2_ref,
             lw_ref, lb_ref, bt_ref, o_ref):
        hs = h_ref[...] + agg_ref[0] + agg_ref[1]
        z = jnp.dot(hs, w1_ref[...], preferred_element_type=jnp.float32) + b1_ref[...]
        z = jnp.maximum(z, 0.0)
        z = jnp.dot(z, w2_ref[...], preferred_element_type=jnp.float32) + b2_ref[...]
        hh = jnp.maximum(z, 0.0)
        y = jnp.dot(hh, lw_ref[...], preferred_element_type=jnp.float32)
        seg = lax.broadcasted_iota(jnp.int32, (bm, B_SEG), 1)
        oh = (bt_ref[...] == seg).astype(jnp.float32)
        contrib = jnp.sum(oh * y, axis=0, keepdims=True)

        @pl.when(pl.program_id(0) == 0)
        def _():
            o_ref[...] = contrib + lb_ref[...]

        @pl.when(pl.program_id(0) != 0)
        def _():
            o_ref[...] = o_ref[...] + contrib

    out = pl.pallas_call(
        body,
        grid=(n // bm,),
        in_specs=[
            pl.BlockSpec((bm, din), lambda i: (i, 0)),
            pl.BlockSpec((2, bm, din), lambda i: (0, i, 0)),
            pl.BlockSpec((din, hdim), lambda i: (0, 0)),
            pl.BlockSpec((1, hdim), lambda i: (0, 0)),
            pl.BlockSpec((hdim, hdim), lambda i: (0, 0)),
            pl.BlockSpec((1, hdim), lambda i: (0, 0)),
            pl.BlockSpec((hdim, 1), lambda i: (0, 0)),
            pl.BlockSpec((1, 1), lambda i: (0, 0)),
            pl.BlockSpec((bm, 1), lambda i: (i, 0)),
        ],
        out_specs=pl.BlockSpec((1, B_SEG), lambda i: (0, 0)),
        out_shape=jax.ShapeDtypeStruct((1, B_SEG), jnp.float32),
    )(h, agg, w1, b1.reshape(1, -1), w2, b2.reshape(1, -1),
      lin_w, lin_b.reshape(1, 1), batch_i32)
    return out[0]


def kernel(x, edge_index, edge_attr, batch,
           W1_0, b1_0, W2_0, b2_0, W1_1, b1_1, W2_1, b2_1,
           W1_2, b1_2, W2_2, b2_2, lin_W, lin_b):
    n, d = x.shape
    e = edge_index.shape[1]
    npad, _ = _acc_pad(n)

    def pad_edges(chunk, nbuf):
        nchunk = -(-e // (NW * chunk))
        nchunk = -(-nchunk // nbuf) * nbuf
        e_pad = NW * nchunk * chunk
        src = edge_index[0]
        dst = edge_index[1]
        if e_pad > e:
            fill = jnp.arange(e_pad - e, dtype=jnp.int32)
            src = jnp.concatenate([src, fill % n])
            dst = jnp.concatenate([dst, n + (fill % (npad - n))])
        return src.reshape(NW, nchunk, chunk), dst.reshape(NW, nchunk, chunk), nchunk

    plans = {128: (112, 2), 64: (128, 4)}
    edge_plans = {dd: (cfg[0], cfg[1]) + pad_edges(*cfg)
                  for dd, cfg in plans.items()}

    batch_i32 = batch.astype(jnp.int32).reshape(n, 1)

    layers = [(W1_0, b1_0, W2_0, b2_0), (W1_1, b1_1, W2_1, b2_1),
              (W1_2, b1_2, W2_2, b2_2)]

    h = x
    for li, (w1, bb1, w2, bb2) in enumerate(layers):
        chunk, nbuf, src_p, dst_p, nchunk = edge_plans[min(h.shape[1], 128)]
        sc_agg = _make_sc_agg(n, h.shape[1], nchunk, chunk, nbuf)
        agg = sc_agg(h, src_p, dst_p)
        if li < 2:
            h = _mlp_layer(h, agg, w1, bb1, w2, bb2)
        else:
            out = _final_layer(h, agg, w1, bb1, w2, bb2, lin_W, lin_b,
                               batch_i32)
    return out
